# baseline (speedup 1.0000x reference)
import jax
import jax.numpy as jnp
from jax.experimental import pallas as pl
from jax.experimental.pallas import tpu as pltpu

_ROW_TILE = 16384


def _linear_relu_body(x_ref, w_ref, b_ref, o_ref):
    acc = jax.lax.dot_general(
        x_ref[...], w_ref[...],
        dimension_numbers=(((1,), (0,)), ((), ())),
        preferred_element_type=jnp.float32,
    )
    o_ref[...] = jnp.maximum(acc + b_ref[0, 0], 0.0).astype(o_ref.dtype)


def kernel(x, weight, bias):
    n, f = x.shape
    w_col = weight.reshape(f, 1).astype(jnp.float32)
    bsc = bias.reshape(1, 1).astype(jnp.float32)

    tn = _ROW_TILE if n > _ROW_TILE else n
    out = pl.pallas_call(
        _linear_relu_body,
        out_shape=jax.ShapeDtypeStruct((n, 1), x.dtype),
        grid=(pl.cdiv(n, tn),),
        in_specs=[
            pl.BlockSpec((tn, f), lambda i: (i, 0)),
            pl.BlockSpec((f, 1), lambda i: (0, 0)),
            pl.BlockSpec(memory_space=pltpu.MemorySpace.SMEM),
        ],
        out_specs=pl.BlockSpec((tn, 1), lambda i: (i, 0)),
        compiler_params=pltpu.CompilerParams(
            dimension_semantics=("parallel",)),
        cost_estimate=pl.CostEstimate(
            flops=2 * n * f, transcendentals=0,
            bytes_accessed=(n * f + n) * jnp.dtype(x.dtype).itemsize),
    )(x, w_col, bsc)
    return out


# speedup vs baseline: 17.7597x; 17.7597x over previous
import jax
import jax.numpy as jnp
from jax.experimental import pallas as pl
from jax.experimental.pallas import tpu as pltpu

_COL_TILE = 32768


def _matvec_cols_body(xt_ref, w_ref, b_ref, o_ref):
    acc = jax.lax.dot_general(
        w_ref[...], xt_ref[...],
        dimension_numbers=(((1,), (0,)), ((), ())),
        preferred_element_type=jnp.float32,
    )
    o_ref[...] = jnp.maximum(acc + b_ref[0, 0], 0.0).astype(o_ref.dtype)


def _rowsum_body(x_ref, w_ref, b_ref, o_ref):
    prod = x_ref[...].astype(jnp.float32) * w_ref[...]
    y = jnp.sum(prod, axis=-1, keepdims=True) + b_ref[0, 0]
    o_ref[...] = jnp.maximum(y, 0.0).astype(o_ref.dtype)


def kernel(x, weight, bias):
    n, f = x.shape
    bsc = bias.reshape(1, 1).astype(jnp.float32)

    if n % 128 == 0 and f <= 512:
        xt = x.T
        w_row = weight.reshape(1, f).astype(jnp.float32)
        tc = _COL_TILE if n > _COL_TILE else n
        out = pl.pallas_call(
            _matvec_cols_body,
            out_shape=jax.ShapeDtypeStruct((1, n), x.dtype),
            grid=(pl.cdiv(n, tc),),
            in_specs=[
                pl.BlockSpec((f, tc), lambda i: (0, i)),
                pl.BlockSpec((1, f), lambda i: (0, 0)),
                pl.BlockSpec(memory_space=pltpu.MemorySpace.SMEM),
            ],
            out_specs=pl.BlockSpec((1, tc), lambda i: (0, i)),
            compiler_params=pltpu.CompilerParams(
                dimension_semantics=("parallel",)),
            cost_estimate=pl.CostEstimate(
                flops=2 * n * f, transcendentals=0,
                bytes_accessed=(n * f + n) * jnp.dtype(x.dtype).itemsize),
        )(xt, w_row, bsc)
        return out.reshape(n, 1)

    w_row = weight.reshape(1, f).astype(jnp.float32)
    tn = min(n, 8192)
    out = pl.pallas_call(
        _rowsum_body,
        out_shape=jax.ShapeDtypeStruct((n, 1), x.dtype),
        grid=(pl.cdiv(n, tn),),
        in_specs=[
            pl.BlockSpec((tn, f), lambda i: (i, 0)),
            pl.BlockSpec((1, f), lambda i: (0, 0)),
            pl.BlockSpec(memory_space=pltpu.MemorySpace.SMEM),
        ],
        out_specs=pl.BlockSpec((tn, 1), lambda i: (i, 0)),
        compiler_params=pltpu.CompilerParams(
            dimension_semantics=("parallel",)),
    )(x, w_row, bsc)
    return out


# speedup vs baseline: 19.1273x; 1.0770x over previous
import jax
import jax.numpy as jnp
from jax.experimental import pallas as pl
from jax.experimental.pallas import tpu as pltpu

_COL_TILE = 65536


def _matvec_cols_body(xt_ref, w_ref, b_ref, o_ref):
    acc = jax.lax.dot_general(
        w_ref[...], xt_ref[...],
        dimension_numbers=(((1,), (0,)), ((), ())),
        preferred_element_type=jnp.float32,
    )
    o_ref[...] = jnp.maximum(acc + b_ref[0, 0], 0.0).astype(o_ref.dtype)


def _rowsum_body(x_ref, w_ref, b_ref, o_ref):
    prod = x_ref[...].astype(jnp.float32) * w_ref[...]
    y = jnp.sum(prod, axis=-1, keepdims=True) + b_ref[0, 0]
    o_ref[...] = jnp.maximum(y, 0.0).astype(o_ref.dtype)


def kernel(x, weight, bias):
    n, f = x.shape
    bsc = bias.reshape(1, 1).astype(jnp.float32)

    if n % 128 == 0 and f <= 512:
        xt = x.T
        w_row = weight.reshape(1, f).astype(jnp.float32)
        tc = _COL_TILE if n > _COL_TILE else n
        out = pl.pallas_call(
            _matvec_cols_body,
            out_shape=jax.ShapeDtypeStruct((1, n), x.dtype),
            grid=(pl.cdiv(n, tc),),
            in_specs=[
                pl.BlockSpec((f, tc), lambda i: (0, i)),
                pl.BlockSpec((1, f), lambda i: (0, 0)),
                pl.BlockSpec(memory_space=pltpu.MemorySpace.SMEM),
            ],
            out_specs=pl.BlockSpec((1, tc), lambda i: (0, i)),
            compiler_params=pltpu.CompilerParams(
                dimension_semantics=("parallel",)),
            cost_estimate=pl.CostEstimate(
                flops=2 * n * f, transcendentals=0,
                bytes_accessed=(n * f + n) * jnp.dtype(x.dtype).itemsize),
        )(xt, w_row, bsc)
        return out.reshape(n, 1)

    w_row = weight.reshape(1, f).astype(jnp.float32)
    tn = min(n, 8192)
    out = pl.pallas_call(
        _rowsum_body,
        out_shape=jax.ShapeDtypeStruct((n, 1), x.dtype),
        grid=(pl.cdiv(n, tn),),
        in_specs=[
            pl.BlockSpec((tn, f), lambda i: (i, 0)),
            pl.BlockSpec((1, f), lambda i: (0, 0)),
            pl.BlockSpec(memory_space=pltpu.MemorySpace.SMEM),
        ],
        out_specs=pl.BlockSpec((tn, 1), lambda i: (i, 0)),
        compiler_params=pltpu.CompilerParams(
            dimension_semantics=("parallel",)),
    )(x, w_row, bsc)
    return out
